# Initial kernel scaffold; baseline (speedup 1.0000x reference)
import jax
import jax.numpy as jnp
from jax.experimental import pallas as pl


def kernel(x, edge_index, W1, b1, W2, b2):
    raise NotImplementedError("write your pallas kernel here")


# speedup vs baseline: 7.9737x; 7.9737x over previous
import functools

import jax
import jax.numpy as jnp
from jax import lax
from jax.experimental import pallas as pl
from jax.experimental.pallas import tpu as pltpu
from jax.experimental.pallas import tpu_sc as plsc

NC = 2
NS = 16
NW = NC * NS
CHUNK = 128


def _relu_body(x_ref, o_ref):
    o_ref[...] = jnp.maximum(x_ref[...], 0.0)


def _row_block(n):
    for b in (1000, 800, 512, 400, 256, 200, 128, 80, 40, 8):
        if n % b == 0:
            return b
    return n


@functools.lru_cache(maxsize=None)
def _make_relu(n, d):
    rb = _row_block(n)
    return pl.pallas_call(
        _relu_body,
        grid=(n // rb,),
        in_specs=[pl.BlockSpec((rb, d), lambda i: (i, 0))],
        out_specs=pl.BlockSpec((rb, d), lambda i: (i, 0)),
        out_shape=jax.ShapeDtypeStruct((n, d), jnp.float32),
    )


def _mlp_body(x_ref, p0_ref, p1_ref, w_ref, b_ref, o_ref, *, scale):
    h = x_ref[...] + p0_ref[...] + p1_ref[...]
    acc = jnp.dot(h, w_ref[...], preferred_element_type=jnp.float32)
    acc = scale * (acc + b_ref[...])
    o_ref[...] = jnp.maximum(acc, 0.0)


@functools.lru_cache(maxsize=None)
def _make_mlp(n, npad, d, scale):
    rb = _row_block(n)
    return pl.pallas_call(
        functools.partial(_mlp_body, scale=scale),
        grid=(n // rb,),
        in_specs=[
            pl.BlockSpec((rb, d), lambda i: (i, 0)),
            pl.BlockSpec((rb, d), lambda i: (i, 0)),
            pl.BlockSpec((rb, d), lambda i: (i, 0)),
            pl.BlockSpec((d, d), lambda i: (0, 0)),
            pl.BlockSpec((1, d), lambda i: (0, 0)),
        ],
        out_specs=pl.BlockSpec((rb, d), lambda i: (i, 0)),
        out_shape=jax.ShapeDtypeStruct((n, d), jnp.float32),
    )


@functools.lru_cache(maxsize=None)
def _make_sc_scatter(n, npad, d, c):
    rows_per_tile = npad // NS
    mesh = plsc.VectorSubcoreMesh(core_axis_name="c", subcore_axis_name="s")

    @functools.partial(
        pl.kernel,
        out_type=jax.ShapeDtypeStruct((NC, npad, d), jnp.float32),
        mesh=mesh,
        scratch_types=[
            pltpu.VMEM((c, CHUNK), jnp.int32),
            pltpu.VMEM((c, CHUNK), jnp.int32),
            pltpu.VMEM((CHUNK, d), jnp.float32),
            pltpu.VMEM_SHARED((npad, d), jnp.float32),
            pltpu.SemaphoreType.DMA,
        ],
    )
    def sc_scatter(r_hbm, src_hbm, dst_hbm, zeros_hbm, out_hbm,
                   src_v, dst_v, rows_v, agg_sh, sem):
        cid = lax.axis_index("c")
        sid = lax.axis_index("s")
        wid = cid * NS + sid
        pltpu.sync_copy(zeros_hbm,
                        agg_sh.at[pl.ds(sid * rows_per_tile, rows_per_tile)])
        pltpu.sync_copy(src_hbm.at[wid], src_v)
        pltpu.sync_copy(dst_hbm.at[wid], dst_v)
        plsc.subcore_barrier()

        def body(j, carry):
            pltpu.async_copy(r_hbm.at[src_v.at[j]], rows_v, sem).wait()
            pltpu.sync_copy(rows_v, agg_sh.at[dst_v.at[j]], add=True)
            return carry

        lax.fori_loop(0, c, body, 0)
        plsc.subcore_barrier()
        pltpu.sync_copy(agg_sh.at[pl.ds(sid * rows_per_tile, rows_per_tile)],
                        out_hbm.at[cid, pl.ds(sid * rows_per_tile, rows_per_tile)])

    return sc_scatter


def kernel(x, edge_index, W1, b1, W2, b2):
    n, d = x.shape
    e = edge_index.shape[1]
    src = edge_index[0].astype(jnp.int32)
    dst = edge_index[1].astype(jnp.int32)

    c = -(-e // (NW * CHUNK))
    ep = NW * c * CHUNK
    npad = ((n + 127) // 128) * 128 + 128
    pad = ep - e
    pad_src = jnp.arange(pad, dtype=jnp.int32) % n
    pad_dst = n + jnp.arange(pad, dtype=jnp.int32) % (npad - n)
    src3 = jnp.concatenate([src, pad_src]).reshape(NW, c, CHUNK)
    dst3 = jnp.concatenate([dst, pad_dst]).reshape(NW, c, CHUNK)
    zeros = jnp.zeros((npad // NS, d), jnp.float32)

    relu = _make_relu(n, d)
    sc_scatter = _make_sc_scatter(n, npad, d, c)
    mlp1 = _make_mlp(n, npad, d, 1.0)
    mlp2 = _make_mlp(n, npad, d, 2.0)
    b1r = b1.reshape(1, d)
    b2r = b2.reshape(1, d)

    r1 = relu(x)
    p = sc_scatter(r1, src3, dst3, zeros)
    out1 = mlp1(x, p[0], p[1], W1, b1r)
    q = sc_scatter(out1, src3, dst3, zeros)
    out = mlp2(out1, q[0], q[1], W2, b2r)
    return out


# speedup vs baseline: 10.7130x; 1.3435x over previous
import functools

import jax
import jax.numpy as jnp
from jax import lax
from jax.experimental import pallas as pl
from jax.experimental.pallas import tpu as pltpu
from jax.experimental.pallas import tpu_sc as plsc

NC = 2
NS = 16
NW = NC * NS
CHUNK = 128
NBUF = 2


def _relu_body(x_ref, o_ref):
    o_ref[...] = jnp.maximum(x_ref[...], 0.0)


def _row_block(n):
    for b in (1000, 800, 512, 400, 256, 200, 128, 80, 40, 8):
        if n % b == 0:
            return b
    return n


@functools.lru_cache(maxsize=None)
def _make_relu(n, d):
    rb = _row_block(n)
    return pl.pallas_call(
        _relu_body,
        grid=(n // rb,),
        in_specs=[pl.BlockSpec((rb, d), lambda i: (i, 0))],
        out_specs=pl.BlockSpec((rb, d), lambda i: (i, 0)),
        out_shape=jax.ShapeDtypeStruct((n, d), jnp.float32),
    )


def _mlp_body(x_ref, p0_ref, p1_ref, w_ref, b_ref, o_ref, *, scale):
    h = x_ref[...] + p0_ref[0] + p1_ref[0]
    acc = jnp.dot(h, w_ref[...], preferred_element_type=jnp.float32)
    acc = scale * (acc + b_ref[...])
    o_ref[...] = jnp.maximum(acc, 0.0)


@functools.lru_cache(maxsize=None)
def _make_mlp(n, npad, d, scale):
    rb = _row_block(n)
    return pl.pallas_call(
        functools.partial(_mlp_body, scale=scale),
        grid=(n // rb,),
        in_specs=[
            pl.BlockSpec((rb, d), lambda i: (i, 0)),
            pl.BlockSpec((1, rb, d), lambda i: (0, i, 0)),
            pl.BlockSpec((1, rb, d), lambda i: (1, i, 0)),
            pl.BlockSpec((d, d), lambda i: (0, 0)),
            pl.BlockSpec((1, d), lambda i: (0, 0)),
        ],
        out_specs=pl.BlockSpec((rb, d), lambda i: (i, 0)),
        out_shape=jax.ShapeDtypeStruct((n, d), jnp.float32),
    )


@functools.lru_cache(maxsize=None)
def _make_sc_scatter(n, npad, d, c, accum_init):
    rows_per_tile = npad // NS
    mesh = plsc.VectorSubcoreMesh(core_axis_name="c", subcore_axis_name="s")
    nbuf = NBUF if (c >= NBUF and c % NBUF == 0) else 1

    @functools.partial(
        pl.kernel,
        out_type=jax.ShapeDtypeStruct((NC, npad, d), jnp.float32),
        mesh=mesh,
        scratch_types=(
            [pltpu.VMEM((c, CHUNK), jnp.int32),
             pltpu.VMEM((c, CHUNK), jnp.int32),
             pltpu.VMEM_SHARED((npad, d), jnp.float32)]
            + [pltpu.VMEM((CHUNK, d), jnp.float32) for _ in range(nbuf)]
            + [pltpu.SemaphoreType.DMA for _ in range(nbuf)]
        ),
    )
    def sc_scatter(r_hbm, src_hbm, dst_hbm, init_hbm, out_hbm,
                   src_v, dst_v, agg_sh, *bufs_and_sems):
        rows = bufs_and_sems[:nbuf]
        gsem = bufs_and_sems[nbuf:]
        cid = lax.axis_index("c")
        sid = lax.axis_index("s")
        wid = cid * NS + sid
        stripe = pl.ds(sid * rows_per_tile, rows_per_tile)
        if accum_init:
            pltpu.sync_copy(init_hbm.at[cid, stripe], agg_sh.at[stripe])
        else:
            pltpu.sync_copy(init_hbm, agg_sh.at[stripe])
        pltpu.sync_copy(src_hbm.at[wid], src_v)
        pltpu.sync_copy(dst_hbm.at[wid], dst_v)
        plsc.subcore_barrier()

        for t in range(nbuf):
            pltpu.async_copy(r_hbm.at[src_v.at[t]], rows[t], gsem[t])

        def step(j, t):
            pltpu.make_async_copy(r_hbm.at[src_v.at[j]], rows[t], gsem[t]).wait()
            pltpu.sync_copy(rows[t], agg_sh.at[dst_v.at[j]], add=True)

        def outer(i, carry):
            j0 = i * nbuf
            for t in range(nbuf):
                step(j0 + t, t)
                pltpu.async_copy(r_hbm.at[src_v.at[j0 + t + nbuf]],
                                 rows[t], gsem[t])
            return carry

        lax.fori_loop(0, c // nbuf - 1, outer, 0)
        for t in range(nbuf):
            step(c - nbuf + t, t)

        plsc.subcore_barrier()
        pltpu.sync_copy(agg_sh.at[stripe], out_hbm.at[cid, stripe])

    return sc_scatter


def kernel(x, edge_index, W1, b1, W2, b2):
    n, d = x.shape
    e = edge_index.shape[1]
    src = edge_index[0].astype(jnp.int32)
    dst = edge_index[1].astype(jnp.int32)

    c = -(-e // (NW * CHUNK))
    c = ((c + 2 * NBUF - 1) // (2 * NBUF)) * (2 * NBUF)
    c2 = c // 2
    ep = NW * c * CHUNK
    npad = ((n + 127) // 128) * 128
    if npad == n:
        npad += 128
    pad = ep - e
    pad_src = jnp.arange(pad, dtype=jnp.int32) % n
    pad_dst = n + jnp.arange(pad, dtype=jnp.int32) % (npad - n)
    src3 = jnp.concatenate([src, pad_src]).reshape(NW, c, CHUNK)
    dst3 = jnp.concatenate([dst, pad_dst]).reshape(NW, c, CHUNK)
    src_a, src_b = src3[:, :c2], src3[:, c2:]
    dst_a, dst_b = dst3[:, :c2], dst3[:, c2:]
    zeros = jnp.zeros((npad // NS, d), jnp.float32)

    relu = _make_relu(n, d)
    sc_a = _make_sc_scatter(n, npad, d, c2, False)
    sc_b = _make_sc_scatter(n, npad, d, c2, True)
    mlp1 = _make_mlp(n, npad, d, 1.0)
    mlp2 = _make_mlp(n, npad, d, 2.0)
    b1r = b1.reshape(1, d)
    b2r = b2.reshape(1, d)

    r1 = relu(x)
    p = sc_b(r1, src_b, dst_b, sc_a(r1, src_a, dst_a, zeros))
    out1 = mlp1(x, p, p, W1, b1r)
    q = sc_b(out1, src_b, dst_b, sc_a(out1, src_a, dst_a, zeros))
    out = mlp2(out1, q, q, W2, b2r)
    return out


# speedup vs baseline: 12.0362x; 1.1235x over previous
import functools

import jax
import jax.numpy as jnp
from jax import lax
from jax.experimental import pallas as pl
from jax.experimental.pallas import tpu as pltpu
from jax.experimental.pallas import tpu_sc as plsc

NC = 2
NS = 16
NW = NC * NS
CHUNK = 128
NBUF = 2


def _relu_body(x_ref, o_ref):
    o_ref[...] = jnp.maximum(x_ref[...], 0.0)


def _row_block(n):
    for b in (1000, 800, 512, 400, 256, 200, 128, 80, 40, 8):
        if n % b == 0:
            return b
    return n


@functools.lru_cache(maxsize=None)
def _make_relu(n, d):
    rb = _row_block(n)
    return pl.pallas_call(
        _relu_body,
        grid=(n // rb,),
        in_specs=[pl.BlockSpec((rb, d), lambda i: (i, 0))],
        out_specs=pl.BlockSpec((rb, d), lambda i: (i, 0)),
        out_shape=jax.ShapeDtypeStruct((n, d), jnp.float32),
    )


def _mlp_body(x_ref, p0_ref, p1_ref, w_ref, b_ref, o_ref, *, scale):
    h = x_ref[...] + p0_ref[0] + p1_ref[0]
    acc = jnp.dot(h, w_ref[...], preferred_element_type=jnp.float32)
    acc = scale * (acc + b_ref[...])
    o_ref[...] = jnp.maximum(acc, 0.0)


@functools.lru_cache(maxsize=None)
def _make_mlp(n, npad, d, scale):
    rb = _row_block(n)
    return pl.pallas_call(
        functools.partial(_mlp_body, scale=scale),
        grid=(n // rb,),
        in_specs=[
            pl.BlockSpec((rb, d), lambda i: (i, 0)),
            pl.BlockSpec((1, rb, d), lambda i: (0, i, 0)),
            pl.BlockSpec((1, rb, d), lambda i: (1, i, 0)),
            pl.BlockSpec((d, d), lambda i: (0, 0)),
            pl.BlockSpec((1, d), lambda i: (0, 0)),
        ],
        out_specs=pl.BlockSpec((rb, d), lambda i: (i, 0)),
        out_shape=jax.ShapeDtypeStruct((n, d), jnp.float32),
    )


@functools.lru_cache(maxsize=None)
def _make_sc_scatter(n, npad, d, c, nphase):
    rows_per_tile = npad // NS
    cp = c // nphase
    mesh = plsc.VectorSubcoreMesh(core_axis_name="c", subcore_axis_name="s")
    nbuf = NBUF if (cp >= NBUF and cp % NBUF == 0) else 1

    @functools.partial(
        pl.kernel,
        out_type=jax.ShapeDtypeStruct((NC, npad, d), jnp.float32),
        mesh=mesh,
        scratch_types=(
            [pltpu.VMEM((cp, CHUNK), jnp.int32),
             pltpu.VMEM((cp, CHUNK), jnp.int32),
             pltpu.VMEM_SHARED((npad, d), jnp.float32)]
            + [pltpu.VMEM((CHUNK, d), jnp.float32) for _ in range(nbuf)]
            + [pltpu.SemaphoreType.DMA for _ in range(nbuf)]
        ),
    )
    def sc_scatter(r_hbm, src_hbm, dst_hbm, zeros_hbm, out_hbm,
                   src_v, dst_v, agg_sh, *bufs_and_sems):
        rows = bufs_and_sems[:nbuf]
        gsem = bufs_and_sems[nbuf:]
        cid = lax.axis_index("c")
        sid = lax.axis_index("s")
        wid = cid * NS + sid
        stripe = pl.ds(sid * rows_per_tile, rows_per_tile)
        pltpu.sync_copy(zeros_hbm, agg_sh.at[stripe])

        def step(j, t):
            pltpu.make_async_copy(r_hbm.at[src_v.at[j]], rows[t], gsem[t]).wait()
            pltpu.sync_copy(rows[t], agg_sh.at[dst_v.at[j]], add=True)

        def outer(i, carry):
            j0 = i * nbuf
            for t in range(nbuf):
                step(j0 + t, t)
                pltpu.async_copy(r_hbm.at[src_v.at[j0 + t + nbuf]],
                                 rows[t], gsem[t])
            return carry

        for ph in range(nphase):
            pltpu.sync_copy(src_hbm.at[wid, pl.ds(ph * cp, cp)], src_v)
            pltpu.sync_copy(dst_hbm.at[wid, pl.ds(ph * cp, cp)], dst_v)
            if ph == 0:
                plsc.subcore_barrier()
            for t in range(nbuf):
                pltpu.async_copy(r_hbm.at[src_v.at[t]], rows[t], gsem[t])
            lax.fori_loop(0, cp // nbuf - 1, outer, 0)
            for t in range(nbuf):
                step(cp - nbuf + t, t)

        plsc.subcore_barrier()
        pltpu.sync_copy(agg_sh.at[stripe], out_hbm.at[cid, stripe])

    return sc_scatter


def kernel(x, edge_index, W1, b1, W2, b2):
    n, d = x.shape
    e = edge_index.shape[1]
    src = edge_index[0].astype(jnp.int32)
    dst = edge_index[1].astype(jnp.int32)

    nphase = 2
    c = -(-e // (NW * CHUNK))
    step = nphase * NBUF
    c = ((c + step - 1) // step) * step
    ep = NW * c * CHUNK
    npad = ((n + 127) // 128) * 128
    if npad == n:
        npad += 128
    pad = ep - e
    pad_src = jnp.arange(pad, dtype=jnp.int32) % n
    pad_dst = n + jnp.arange(pad, dtype=jnp.int32) % (npad - n)
    src3 = jnp.concatenate([src, pad_src]).reshape(NW, c, CHUNK)
    dst3 = jnp.concatenate([dst, pad_dst]).reshape(NW, c, CHUNK)
    zeros = jnp.zeros((npad // NS, d), jnp.float32)

    relu = _make_relu(n, d)
    sc_scatter = _make_sc_scatter(n, npad, d, c, nphase)
    mlp1 = _make_mlp(n, npad, d, 1.0)
    mlp2 = _make_mlp(n, npad, d, 2.0)
    b1r = b1.reshape(1, d)
    b2r = b2.reshape(1, d)

    r1 = relu(x)
    p = sc_scatter(r1, src3, dst3, zeros)
    out1 = mlp1(x, p, p, W1, b1r)
    q = sc_scatter(out1, src3, dst3, zeros)
    out = mlp2(out1, q, q, W2, b2r)
    return out


# speedup vs baseline: 12.4708x; 1.0361x over previous
import functools

import jax
import jax.numpy as jnp
from jax import lax
from jax.experimental import pallas as pl
from jax.experimental.pallas import tpu as pltpu
from jax.experimental.pallas import tpu_sc as plsc

NC = 2
NS = 16
NW = NC * NS
CHUNK = 64
NBUF = 4


def _relu_body(x_ref, o_ref):
    o_ref[...] = jnp.maximum(x_ref[...], 0.0)


def _row_block(n):
    for b in (1000, 800, 512, 400, 256, 200, 128, 80, 40, 8):
        if n % b == 0:
            return b
    return n


@functools.lru_cache(maxsize=None)
def _make_relu(n, d):
    rb = _row_block(n)
    return pl.pallas_call(
        _relu_body,
        grid=(n // rb,),
        in_specs=[pl.BlockSpec((rb, d), lambda i: (i, 0))],
        out_specs=pl.BlockSpec((rb, d), lambda i: (i, 0)),
        out_shape=jax.ShapeDtypeStruct((n, d), jnp.float32),
    )


def _mlp_body(x_ref, p0_ref, p1_ref, w_ref, b_ref, o_ref, *, scale):
    h = x_ref[...] + p0_ref[0] + p1_ref[0]
    acc = jnp.dot(h, w_ref[...], preferred_element_type=jnp.float32)
    acc = scale * (acc + b_ref[...])
    o_ref[...] = jnp.maximum(acc, 0.0)


@functools.lru_cache(maxsize=None)
def _make_mlp(n, npad, d, scale):
    rb = _row_block(n)
    return pl.pallas_call(
        functools.partial(_mlp_body, scale=scale),
        grid=(n // rb,),
        in_specs=[
            pl.BlockSpec((rb, d), lambda i: (i, 0)),
            pl.BlockSpec((1, rb, d), lambda i: (0, i, 0)),
            pl.BlockSpec((1, rb, d), lambda i: (1, i, 0)),
            pl.BlockSpec((d, d), lambda i: (0, 0)),
            pl.BlockSpec((1, d), lambda i: (0, 0)),
        ],
        out_specs=pl.BlockSpec((rb, d), lambda i: (i, 0)),
        out_shape=jax.ShapeDtypeStruct((n, d), jnp.float32),
    )


@functools.lru_cache(maxsize=None)
def _make_sc_scatter(n, npad, d, c, nphase):
    rows_per_tile = npad // NS
    cp = c // nphase
    mesh = plsc.VectorSubcoreMesh(core_axis_name="c", subcore_axis_name="s")
    nbuf = NBUF if (cp >= NBUF and cp % NBUF == 0) else 1

    @functools.partial(
        pl.kernel,
        out_type=jax.ShapeDtypeStruct((NC, npad, d), jnp.float32),
        mesh=mesh,
        scratch_types=(
            [pltpu.VMEM((cp, CHUNK), jnp.int32),
             pltpu.VMEM((cp, CHUNK), jnp.int32),
             pltpu.VMEM_SHARED((npad, d), jnp.float32)]
            + [pltpu.VMEM((CHUNK, d), jnp.float32) for _ in range(nbuf)]
            + [pltpu.SemaphoreType.DMA for _ in range(nbuf)]
        ),
    )
    def sc_scatter(r_hbm, src_hbm, dst_hbm, zeros_hbm, out_hbm,
                   src_v, dst_v, agg_sh, *bufs_and_sems):
        rows = bufs_and_sems[:nbuf]
        gsem = bufs_and_sems[nbuf:]
        cid = lax.axis_index("c")
        sid = lax.axis_index("s")
        wid = cid * NS + sid
        stripe = pl.ds(sid * rows_per_tile, rows_per_tile)
        pltpu.sync_copy(zeros_hbm, agg_sh.at[stripe])
        plsc.subcore_barrier()

        def step(j, t):
            pltpu.make_async_copy(r_hbm.at[src_v.at[j]], rows[t], gsem[t]).wait()
            pltpu.sync_copy(rows[t], agg_sh.at[dst_v.at[j]], add=True)

        def outer(i, carry):
            j0 = i * nbuf
            for t in range(nbuf):
                step(j0 + t, t)
                pltpu.async_copy(r_hbm.at[src_v.at[j0 + t + nbuf]],
                                 rows[t], gsem[t])
            return carry

        def phase(ph, carry):
            pltpu.sync_copy(src_hbm.at[wid, pl.ds(ph * cp, cp)], src_v)
            pltpu.sync_copy(dst_hbm.at[wid, pl.ds(ph * cp, cp)], dst_v)
            for t in range(nbuf):
                pltpu.async_copy(r_hbm.at[src_v.at[t]], rows[t], gsem[t])
            lax.fori_loop(0, cp // nbuf - 1, outer, 0)
            for t in range(nbuf):
                step(cp - nbuf + t, t)
            return carry

        lax.fori_loop(0, nphase, phase, 0)

        plsc.subcore_barrier()
        pltpu.sync_copy(agg_sh.at[stripe], out_hbm.at[cid, stripe])

    return sc_scatter


def kernel(x, edge_index, W1, b1, W2, b2):
    n, d = x.shape
    e = edge_index.shape[1]
    src = edge_index[0].astype(jnp.int32)
    dst = edge_index[1].astype(jnp.int32)

    nphase = 4
    c = -(-e // (NW * CHUNK))
    step = nphase * NBUF
    c = ((c + step - 1) // step) * step
    ep = NW * c * CHUNK
    npad = ((n + 127) // 128) * 128
    if npad == n:
        npad += 128
    pad = ep - e
    pad_src = jnp.arange(pad, dtype=jnp.int32) % n
    pad_dst = n + jnp.arange(pad, dtype=jnp.int32) % (npad - n)
    src3 = jnp.concatenate([src, pad_src]).reshape(NW, c, CHUNK)
    dst3 = jnp.concatenate([dst, pad_dst]).reshape(NW, c, CHUNK)
    zeros = jnp.zeros((npad // NS, d), jnp.float32)

    relu = _make_relu(n, d)
    sc_scatter = _make_sc_scatter(n, npad, d, c, nphase)
    mlp1 = _make_mlp(n, npad, d, 1.0)
    mlp2 = _make_mlp(n, npad, d, 2.0)
    b1r = b1.reshape(1, d)
    b2r = b2.reshape(1, d)

    r1 = relu(x)
    p = sc_scatter(r1, src3, dst3, zeros)
    out1 = mlp1(x, p, p, W1, b1r)
    q = sc_scatter(out1, src3, dst3, zeros)
    out = mlp2(out1, q, q, W2, b2r)
    return out


# speedup vs baseline: 13.3941x; 1.0740x over previous
import functools

import jax
import jax.numpy as jnp
from jax import lax
from jax.experimental import pallas as pl
from jax.experimental.pallas import tpu as pltpu
from jax.experimental.pallas import tpu_sc as plsc

NC = 2
NS = 16
NW = NC * NS
CHUNK = 64
NBUF = 4


def _relu_body(x_ref, o_ref):
    o_ref[...] = jnp.maximum(x_ref[...], 0.0)


def _row_block(n):
    for b in (1000, 800, 512, 400, 256, 200, 128, 80, 40, 8):
        if n % b == 0:
            return b
    return n


@functools.lru_cache(maxsize=None)
def _make_relu(n, d):
    rb = _row_block(n)
    return pl.pallas_call(
        _relu_body,
        grid=(n // rb,),
        in_specs=[pl.BlockSpec((rb, d), lambda i: (i, 0))],
        out_specs=pl.BlockSpec((rb, d), lambda i: (i, 0)),
        out_shape=jax.ShapeDtypeStruct((n, d), jnp.float32),
    )


def _mlp_body(x_ref, p0_ref, p1_ref, w_ref, b_ref, o_ref, *, scale):
    h = x_ref[...] + p0_ref[0] + p1_ref[0]
    acc = jnp.dot(h, w_ref[...], preferred_element_type=jnp.float32)
    acc = scale * (acc + b_ref[...])
    o_ref[...] = jnp.maximum(acc, 0.0)


@functools.lru_cache(maxsize=None)
def _make_mlp(n, npad, d, scale):
    rb = _row_block(n)
    return pl.pallas_call(
        functools.partial(_mlp_body, scale=scale),
        grid=(n // rb,),
        in_specs=[
            pl.BlockSpec((rb, d), lambda i: (i, 0)),
            pl.BlockSpec((1, rb, d), lambda i: (0, i, 0)),
            pl.BlockSpec((1, rb, d), lambda i: (1, i, 0)),
            pl.BlockSpec((d, d), lambda i: (0, 0)),
            pl.BlockSpec((1, d), lambda i: (0, 0)),
        ],
        out_specs=pl.BlockSpec((rb, d), lambda i: (i, 0)),
        out_shape=jax.ShapeDtypeStruct((n, d), jnp.float32),
    )


G = 8


@functools.lru_cache(maxsize=None)
def _make_sc_scatter(n, npad, d, c):
    rows_per_tile = npad // NS
    nslab = c // G
    npair = nslab // 2
    mesh = plsc.VectorSubcoreMesh(core_axis_name="c", subcore_axis_name="s")

    @functools.partial(
        pl.kernel,
        out_type=jax.ShapeDtypeStruct((NC, npad, d), jnp.float32),
        mesh=mesh,
        scratch_types=(
            [pltpu.VMEM((G, CHUNK), jnp.int32),
             pltpu.VMEM((G, CHUNK), jnp.int32),
             pltpu.VMEM((G, CHUNK), jnp.int32),
             pltpu.VMEM((G, CHUNK), jnp.int32),
             pltpu.VMEM_SHARED((npad, d), jnp.float32)]
            + [pltpu.VMEM((CHUNK, d), jnp.float32) for _ in range(NBUF)]
            + [pltpu.SemaphoreType.DMA for _ in range(NBUF)]
            + [pltpu.SemaphoreType.DMA, pltpu.SemaphoreType.DMA]
        ),
    )
    def sc_scatter(r_hbm, src_hbm, dst_hbm, zeros_hbm, out_hbm,
                   sa_src, sa_dst, sb_src, sb_dst, agg_sh, *bufs_and_sems):
        rows = bufs_and_sems[:NBUF]
        gsem = bufs_and_sems[NBUF:2 * NBUF]
        isem = bufs_and_sems[2 * NBUF:]
        srcs = (sa_src, sb_src)
        dsts = (sa_dst, sb_dst)
        cid = lax.axis_index("c")
        sid = lax.axis_index("s")
        wid = cid * NS + sid
        stripe = pl.ds(sid * rows_per_tile, rows_per_tile)
        pltpu.sync_copy(zeros_hbm, agg_sh.at[stripe])
        plsc.subcore_barrier()

        def fetch_slab(s, par, sync):
            if sync:
                pltpu.sync_copy(src_hbm.at[wid, s], srcs[par])
                pltpu.sync_copy(dst_hbm.at[wid, s], dsts[par])
            else:
                pltpu.async_copy(src_hbm.at[wid, s], srcs[par], isem[par])
                pltpu.async_copy(dst_hbm.at[wid, s], dsts[par], isem[par])

        def wait_slab(s, par):
            pltpu.make_async_copy(src_hbm.at[wid, s], srcs[par],
                                  isem[par]).wait()
            pltpu.make_async_copy(dst_hbm.at[wid, s], dsts[par],
                                  isem[par]).wait()

        def do_slab(s, par, last):
            for jj in range(G):
                t = jj % NBUF
                if jj == G - NBUF and not last:
                    wait_slab(s + 1, 1 - par)
                pltpu.make_async_copy(r_hbm.at[srcs[par].at[jj]],
                                      rows[t], gsem[t]).wait()
                pltpu.sync_copy(rows[t], agg_sh.at[dsts[par].at[jj]],
                                add=True)
                if jj < G - NBUF:
                    pltpu.async_copy(r_hbm.at[srcs[par].at[jj + NBUF]],
                                     rows[t], gsem[t])
                elif not last:
                    pltpu.async_copy(
                        r_hbm.at[srcs[1 - par].at[jj + NBUF - G]],
                        rows[t], gsem[t])

        fetch_slab(0, 0, True)
        fetch_slab(1, 1, False)
        for t in range(NBUF):
            pltpu.async_copy(r_hbm.at[sa_src.at[t]], rows[t], gsem[t])

        def pair(k, carry):
            s = 2 * k
            do_slab(s, 0, False)
            fetch_slab(s + 2, 0, False)
            do_slab(s + 1, 1, False)
            fetch_slab(s + 3, 1, False)
            return carry

        lax.fori_loop(0, npair - 1, pair, 0)
        do_slab(nslab - 2, 0, False)
        do_slab(nslab - 1, 1, True)

        plsc.subcore_barrier()
        pltpu.sync_copy(agg_sh.at[stripe], out_hbm.at[cid, stripe])

    return sc_scatter


def kernel(x, edge_index, W1, b1, W2, b2):
    n, d = x.shape
    e = edge_index.shape[1]
    src = edge_index[0].astype(jnp.int32)
    dst = edge_index[1].astype(jnp.int32)

    c = -(-e // (NW * CHUNK))
    c = ((c + 2 * G - 1) // (2 * G)) * (2 * G)
    ep = NW * c * CHUNK
    npad = ((n + 127) // 128) * 128
    if npad == n:
        npad += 128
    pad = ep - e
    pad_src = jnp.arange(pad, dtype=jnp.int32) % n
    pad_dst = n + jnp.arange(pad, dtype=jnp.int32) % (npad - n)
    src3 = jnp.concatenate([src, pad_src]).reshape(NW, c // G, G, CHUNK)
    dst3 = jnp.concatenate([dst, pad_dst]).reshape(NW, c // G, G, CHUNK)
    zeros = jnp.zeros((npad // NS, d), jnp.float32)

    relu = _make_relu(n, d)
    sc_scatter = _make_sc_scatter(n, npad, d, c)
    mlp1 = _make_mlp(n, npad, d, 1.0)
    mlp2 = _make_mlp(n, npad, d, 2.0)
    b1r = b1.reshape(1, d)
    b2r = b2.reshape(1, d)

    r1 = relu(x)
    p = sc_scatter(r1, src3, dst3, zeros)
    out1 = mlp1(x, p, p, W1, b1r)
    q = sc_scatter(out1, src3, dst3, zeros)
    out = mlp2(out1, q, q, W2, b2r)
    return out


# speedup vs baseline: 13.4655x; 1.0053x over previous
import functools

import jax
import jax.numpy as jnp
from jax import lax
from jax.experimental import pallas as pl
from jax.experimental.pallas import tpu as pltpu
from jax.experimental.pallas import tpu_sc as plsc

NC = 2
NS = 16
NW = NC * NS
CHUNK = 64
NBUF = 5


def _relu_body(x_ref, o_ref):
    o_ref[...] = jnp.maximum(x_ref[...], 0.0)


def _row_block(n):
    for b in (1000, 800, 512, 400, 256, 200, 128, 80, 40, 8):
        if n % b == 0:
            return b
    return n


@functools.lru_cache(maxsize=None)
def _make_relu(n, d):
    rb = _row_block(n)
    return pl.pallas_call(
        _relu_body,
        grid=(n // rb,),
        in_specs=[pl.BlockSpec((rb, d), lambda i: (i, 0))],
        out_specs=pl.BlockSpec((rb, d), lambda i: (i, 0)),
        out_shape=jax.ShapeDtypeStruct((n, d), jnp.float32),
    )


def _mlp_body(x_ref, p0_ref, p1_ref, w_ref, b_ref, o_ref, *, scale):
    h = x_ref[...] + p0_ref[0] + p1_ref[0]
    acc = jnp.dot(h, w_ref[...], preferred_element_type=jnp.float32)
    acc = scale * (acc + b_ref[...])
    o_ref[...] = jnp.maximum(acc, 0.0)


@functools.lru_cache(maxsize=None)
def _make_mlp(n, npad, d, scale):
    rb = _row_block(n)
    return pl.pallas_call(
        functools.partial(_mlp_body, scale=scale),
        grid=(n // rb,),
        in_specs=[
            pl.BlockSpec((rb, d), lambda i: (i, 0)),
            pl.BlockSpec((1, rb, d), lambda i: (0, i, 0)),
            pl.BlockSpec((1, rb, d), lambda i: (1, i, 0)),
            pl.BlockSpec((d, d), lambda i: (0, 0)),
            pl.BlockSpec((1, d), lambda i: (0, 0)),
        ],
        out_specs=pl.BlockSpec((rb, d), lambda i: (i, 0)),
        out_shape=jax.ShapeDtypeStruct((n, d), jnp.float32),
    )


G = 10


@functools.lru_cache(maxsize=None)
def _make_sc_scatter(n, npad, d, c):
    rows_per_tile = npad // NS
    nslab = c // G
    npair = nslab // 2
    mesh = plsc.VectorSubcoreMesh(core_axis_name="c", subcore_axis_name="s")

    @functools.partial(
        pl.kernel,
        out_type=jax.ShapeDtypeStruct((NC, npad, d), jnp.float32),
        mesh=mesh,
        scratch_types=(
            [pltpu.VMEM((G, CHUNK), jnp.int32),
             pltpu.VMEM((G, CHUNK), jnp.int32),
             pltpu.VMEM((G, CHUNK), jnp.int32),
             pltpu.VMEM((G, CHUNK), jnp.int32),
             pltpu.VMEM_SHARED((npad, d), jnp.float32)]
            + [pltpu.VMEM((CHUNK, d), jnp.float32) for _ in range(NBUF)]
            + [pltpu.SemaphoreType.DMA for _ in range(NBUF)]
            + [pltpu.SemaphoreType.DMA, pltpu.SemaphoreType.DMA]
        ),
    )
    def sc_scatter(r_hbm, src_hbm, dst_hbm, zeros_hbm, out_hbm,
                   sa_src, sa_dst, sb_src, sb_dst, agg_sh, *bufs_and_sems):
        rows = bufs_and_sems[:NBUF]
        gsem = bufs_and_sems[NBUF:2 * NBUF]
        isem = bufs_and_sems[2 * NBUF:]
        srcs = (sa_src, sb_src)
        dsts = (sa_dst, sb_dst)
        cid = lax.axis_index("c")
        sid = lax.axis_index("s")
        wid = cid * NS + sid
        stripe = pl.ds(sid * rows_per_tile, rows_per_tile)
        pltpu.sync_copy(zeros_hbm, agg_sh.at[stripe])
        plsc.subcore_barrier()

        def fetch_slab(s, par, sync):
            if sync:
                pltpu.sync_copy(src_hbm.at[wid, s], srcs[par])
                pltpu.sync_copy(dst_hbm.at[wid, s], dsts[par])
            else:
                pltpu.async_copy(src_hbm.at[wid, s], srcs[par], isem[par])
                pltpu.async_copy(dst_hbm.at[wid, s], dsts[par], isem[par])

        def wait_slab(s, par):
            pltpu.make_async_copy(src_hbm.at[wid, s], srcs[par],
                                  isem[par]).wait()
            pltpu.make_async_copy(dst_hbm.at[wid, s], dsts[par],
                                  isem[par]).wait()

        def do_slab(s, par, last):
            for jj in range(G):
                t = jj % NBUF
                if jj == G - NBUF and not last:
                    wait_slab(s + 1, 1 - par)
                pltpu.make_async_copy(r_hbm.at[srcs[par].at[jj]],
                                      rows[t], gsem[t]).wait()
                pltpu.sync_copy(rows[t], agg_sh.at[dsts[par].at[jj]],
                                add=True)
                if jj < G - NBUF:
                    pltpu.async_copy(r_hbm.at[srcs[par].at[jj + NBUF]],
                                     rows[t], gsem[t])
                elif not last:
                    pltpu.async_copy(
                        r_hbm.at[srcs[1 - par].at[jj + NBUF - G]],
                        rows[t], gsem[t])

        fetch_slab(0, 0, True)
        fetch_slab(1, 1, False)
        for t in range(NBUF):
            pltpu.async_copy(r_hbm.at[sa_src.at[t]], rows[t], gsem[t])

        def pair(k, carry):
            s = 2 * k
            do_slab(s, 0, False)
            fetch_slab(s + 2, 0, False)
            do_slab(s + 1, 1, False)
            fetch_slab(s + 3, 1, False)
            return carry

        lax.fori_loop(0, npair - 1, pair, 0)
        do_slab(nslab - 2, 0, False)
        do_slab(nslab - 1, 1, True)

        plsc.subcore_barrier()
        pltpu.sync_copy(agg_sh.at[stripe], out_hbm.at[cid, stripe])

    return sc_scatter


def kernel(x, edge_index, W1, b1, W2, b2):
    n, d = x.shape
    e = edge_index.shape[1]
    src = edge_index[0].astype(jnp.int32)
    dst = edge_index[1].astype(jnp.int32)

    c = -(-e // (NW * CHUNK))
    c = ((c + 2 * G - 1) // (2 * G)) * (2 * G)
    ep = NW * c * CHUNK
    npad = ((n + 127) // 128) * 128
    if npad == n:
        npad += 128
    pad = ep - e
    pad_src = jnp.arange(pad, dtype=jnp.int32) % n
    pad_dst = n + jnp.arange(pad, dtype=jnp.int32) % (npad - n)
    src3 = jnp.concatenate([src, pad_src]).reshape(NW, c // G, G, CHUNK)
    dst3 = jnp.concatenate([dst, pad_dst]).reshape(NW, c // G, G, CHUNK)
    zeros = jnp.zeros((npad // NS, d), jnp.float32)

    relu = _make_relu(n, d)
    sc_scatter = _make_sc_scatter(n, npad, d, c)
    mlp1 = _make_mlp(n, npad, d, 1.0)
    mlp2 = _make_mlp(n, npad, d, 2.0)
    b1r = b1.reshape(1, d)
    b2r = b2.reshape(1, d)

    r1 = relu(x)
    p = sc_scatter(r1, src3, dst3, zeros)
    out1 = mlp1(x, p, p, W1, b1r)
    q = sc_scatter(out1, src3, dst3, zeros)
    out = mlp2(out1, q, q, W2, b2r)
    return out
